# Initial kernel scaffold; baseline (speedup 1.0000x reference)
import jax
import jax.numpy as jnp
from jax.experimental import pallas as pl


def kernel(x, a, e, W_pre, b_pre, Wk1, bk1, root1, b1, Wk2, bk2, root2, b2, Wk3, bk3, root3, b3, W_d1, b_d1, W_d2, b_d2, W_zm, b_zm, W_zl, b_zl, eps):
    raise NotImplementedError("write your pallas kernel here")


# speedup vs baseline: 28.4495x; 28.4495x over previous
import functools

import jax
import jax.numpy as jnp
from jax.experimental import pallas as pl
from jax.experimental.pallas import tpu as pltpu

_B, _N, _S = 16, 64, 16


def _leaky(z):
    return jnp.where(z >= 0, z, 0.05 * z)


def _dot(a, b):
    return jnp.dot(a, b, preferred_element_type=jnp.float32)


def _body(x_ref, a_ref, e_ref, W_pre_ref, b_pre_ref,
          V1_ref, Bk1_ref, root1_ref, b1_ref,
          V2_ref, Bk2_ref, root2_ref, b2_ref,
          V3_ref, Bk3_ref, root3_ref, b3_ref,
          W_d1_ref, b_d1_ref, W_d2_ref, b_d2_ref,
          W_zm_ref, b_zm_ref, W_zl_ref, b_zl_ref, eps_ref,
          out_ref, pooled_ref):
    i = pl.program_id(0)
    xb = x_ref[0]
    ab = a_ref[0]
    h = xb[:, :32]
    mask = xb[:, 32:33]

    h = _leaky(_dot(h, W_pre_ref[...]) + b_pre_ref[...])

    def ecc(h, V_ref, Bk_ref, root_ref, b_ref):
        acc = _dot(ab, _dot(h, Bk_ref[...]))
        acc = acc + _dot(h, root_ref[...]) + b_ref[...]
        for s in range(_S):
            acc = acc + _dot(ab * e_ref[0, s], _dot(h, V_ref[s]))
        return _leaky(acc)

    h = ecc(h, V1_ref, Bk1_ref, root1_ref, b1_ref)
    h = ecc(h, V2_ref, Bk2_ref, root2_ref, b2_ref)
    h = ecc(h, V3_ref, Bk3_ref, root3_ref, b3_ref)

    pooled_ref[pl.ds(i, 1), :] = jnp.sum(h * mask, axis=0, keepdims=True)

    @pl.when(i == _B - 1)
    def _head():
        p = pooled_ref[...]
        d1 = _leaky(_dot(p, W_d1_ref[...]) + b_d1_ref[...])
        d2 = _leaky(_dot(d1, W_d2_ref[...]) + b_d2_ref[...])
        zm = _dot(d2, W_zm_ref[...]) + b_zm_ref[...]
        zl = _dot(d2, W_zl_ref[...]) + b_zl_ref[...]
        out_ref[...] = zm + jnp.exp(0.5 * zl) * eps_ref[...]


@jax.jit
def _run(x, a, e_p, W_pre, b_pre, V1, Bk1, root1, b1, V2, Bk2, root2, b2,
         V3, Bk3, root3, b3, W_d1, b_d1, W_d2, b_d2, W_zm, b_zm, W_zl, b_zl,
         eps):
    batch3 = lambda shp: pl.BlockSpec(shp, lambda i: (i,) + (0,) * (len(shp) - 1))
    full = lambda shp: pl.BlockSpec(shp, lambda i: (0,) * len(shp))
    in_specs = [
        batch3((1, _N, 33)),
        batch3((1, _N, _N)),
        batch3((1, _S, _N, _N)),
        full((32, 16)), full((1, 16)),
        full((_S, 16, 32)), full((16, 32)), full((16, 32)), full((1, 32)),
        full((_S, 32, 32)), full((32, 32)), full((32, 32)), full((1, 32)),
        full((_S, 32, 32)), full((32, 32)), full((32, 32)), full((1, 32)),
        full((32, 256)), full((1, 256)),
        full((256, 256)), full((1, 256)),
        full((256, 64)), full((1, 64)),
        full((256, 64)), full((1, 64)),
        full((_B, 64)),
    ]
    return pl.pallas_call(
        _body,
        grid=(_B,),
        in_specs=in_specs,
        out_specs=full((_B, 64)),
        out_shape=jax.ShapeDtypeStruct((_B, 64), jnp.float32),
        scratch_shapes=[pltpu.VMEM((_B, 32), jnp.float32)],
    )(x, a, e_p, W_pre, b_pre, V1, Bk1, root1, b1, V2, Bk2, root2, b2,
      V3, Bk3, root3, b3, W_d1, b_d1, W_d2, b_d2, W_zm, b_zm, W_zl, b_zl, eps)


def kernel(x, a, e, W_pre, b_pre, Wk1, bk1, root1, b1, Wk2, bk2, root2, b2,
           Wk3, bk3, root3, b3, W_d1, b_d1, W_d2, b_d2, W_zm, b_zm,
           W_zl, b_zl, eps):
    e_p = e.transpose(0, 3, 1, 2)
    V1 = Wk1.reshape(_S, 32, 16).transpose(0, 2, 1)
    V2 = Wk2.reshape(_S, 32, 32).transpose(0, 2, 1)
    V3 = Wk3.reshape(_S, 32, 32).transpose(0, 2, 1)
    Bk1 = bk1.reshape(32, 16).T
    Bk2 = bk2.reshape(32, 32).T
    Bk3 = bk3.reshape(32, 32).T
    row = lambda v: v.reshape(1, -1)
    return _run(x, a, e_p, W_pre, row(b_pre), V1, Bk1, root1, row(b1),
                V2, Bk2, root2, row(b2), V3, Bk3, root3, row(b3),
                W_d1, row(b_d1), W_d2, row(b_d2), W_zm, row(b_zm),
                W_zl, row(b_zl), eps)


# speedup vs baseline: 30.2961x; 1.0649x over previous
import jax
import jax.numpy as jnp
from jax.experimental import pallas as pl

_B, _N, _S = 16, 64, 16


def _leaky(z):
    return jnp.where(z >= 0, z, 0.05 * z)


def _dot(a, b):
    return jnp.dot(a, b, preferred_element_type=jnp.float32)


def _body(x_ref, a_ref, e_ref, W_pre_ref, b_pre_ref,
          V1_ref, Bk1_ref, root1_ref, b1_ref,
          V2_ref, Bk2_ref, root2_ref, b2_ref,
          V3_ref, Bk3_ref, root3_ref, b3_ref,
          W_d1_ref, b_d1_ref, W_d2_ref, b_d2_ref,
          W_zm_ref, b_zm_ref, W_zl_ref, b_zl_ref, eps_ref,
          out_ref):
    pooled_rows = []
    for b in range(_B):
        xb = x_ref[b]
        ab = a_ref[b]
        h = xb[:, :32]
        mask = xb[:, 32:33]

        h = _leaky(_dot(h, W_pre_ref[...]) + b_pre_ref[...])

        def ecc(h, V_ref, Bk_ref, root_ref, b_ref):
            acc = _dot(ab, _dot(h, Bk_ref[...]))
            acc = acc + _dot(h, root_ref[...]) + b_ref[...]
            for s in range(_S):
                acc = acc + _dot(ab * e_ref[b, s], _dot(h, V_ref[s]))
            return _leaky(acc)

        h = ecc(h, V1_ref, Bk1_ref, root1_ref, b1_ref)
        h = ecc(h, V2_ref, Bk2_ref, root2_ref, b2_ref)
        h = ecc(h, V3_ref, Bk3_ref, root3_ref, b3_ref)
        pooled_rows.append(jnp.sum(h * mask, axis=0, keepdims=True))

    p = jnp.concatenate(pooled_rows, axis=0)
    d1 = _leaky(_dot(p, W_d1_ref[...]) + b_d1_ref[...])
    d2 = _leaky(_dot(d1, W_d2_ref[...]) + b_d2_ref[...])
    zm = _dot(d2, W_zm_ref[...]) + b_zm_ref[...]
    zl = _dot(d2, W_zl_ref[...]) + b_zl_ref[...]
    out_ref[...] = zm + jnp.exp(0.5 * zl) * eps_ref[...]


@jax.jit
def _run(x, a, e_p, W_pre, b_pre, V1, Bk1, root1, b1, V2, Bk2, root2, b2,
         V3, Bk3, root3, b3, W_d1, b_d1, W_d2, b_d2, W_zm, b_zm, W_zl, b_zl,
         eps):
    return pl.pallas_call(
        _body,
        out_shape=jax.ShapeDtypeStruct((_B, 64), jnp.float32),
    )(x, a, e_p, W_pre, b_pre, V1, Bk1, root1, b1, V2, Bk2, root2, b2,
      V3, Bk3, root3, b3, W_d1, b_d1, W_d2, b_d2, W_zm, b_zm, W_zl, b_zl, eps)


def kernel(x, a, e, W_pre, b_pre, Wk1, bk1, root1, b1, Wk2, bk2, root2, b2,
           Wk3, bk3, root3, b3, W_d1, b_d1, W_d2, b_d2, W_zm, b_zm,
           W_zl, b_zl, eps):
    e_p = e.transpose(0, 3, 1, 2)
    V1 = Wk1.reshape(_S, 32, 16).transpose(0, 2, 1)
    V2 = Wk2.reshape(_S, 32, 32).transpose(0, 2, 1)
    V3 = Wk3.reshape(_S, 32, 32).transpose(0, 2, 1)
    Bk1 = bk1.reshape(32, 16).T
    Bk2 = bk2.reshape(32, 32).T
    Bk3 = bk3.reshape(32, 32).T
    row = lambda v: v.reshape(1, -1)
    return _run(x, a, e_p, W_pre, row(b_pre), V1, Bk1, root1, row(b1),
                V2, Bk2, root2, row(b2), V3, Bk3, root3, row(b3),
                W_d1, row(b_d1), W_d2, row(b_d2), W_zm, row(b_zm),
                W_zl, row(b_zl), eps)
